# baseline (speedup 1.0000x reference)
import jax
import jax.numpy as jnp
from jax.experimental import pallas as pl


def _bdiag(w):
    z = jnp.zeros_like(w)
    return jnp.concatenate([jnp.concatenate([w, z], axis=1),
                            jnp.concatenate([z, w], axis=1)], axis=0)


def _tile2(v):
    return jnp.concatenate([v, v], axis=1)


def _gcn_fused_kernel(xe_ref, xev_ref, xnc_ref, ee_ref, ne_ref, wd_ref,
                      wco_ref, wm_ref, bm_ref, wn_ref, bn_ref, wc_ref,
                      bc_ref, out_ref):
    N = xe_ref.shape[1]
    H = ne_ref.shape[1]
    f32 = jnp.float32

    tab = jnp.concatenate([ee_ref[...], wd_ref[...]], axis=0)
    tab_hi = tab.astype(jnp.bfloat16).astype(jnp.float32)
    tab_lo = tab - tab_hi
    wd_hi = tab_hi[3:4]
    w9 = jnp.concatenate([tab_hi, tab_lo, wd_hi], axis=0)
    w18 = _bdiag(w9)
    wco2 = _bdiag(wco_ref[...])
    wm = wm_ref[...]
    we2 = _bdiag(wm[:H])
    ws2 = _bdiag(wm[H:2 * H])
    wsd2 = _bdiag(wm[2 * H:])
    bm2 = _tile2(bm_ref[...])
    wn2 = _bdiag(wn_ref[...])
    bn2 = _tile2(bn_ref[...])
    wc = wc_ref[...]
    wce2 = _bdiag(wc[:H])
    wcs2 = _bdiag(wc[H:2 * H])
    wcd2 = _bdiag(wc[2 * H:]) + _tile2(bc_ref[...])
    ne2 = _tile2(ne_ref[...])

    for q in (0, 2):
        xe0 = xe_ref[q]
        xe1 = xe_ref[q + 1]
        xv0 = xev_ref[q]
        xv1 = xev_ref[q + 1]
        xh0 = xv0.astype(jnp.bfloat16).astype(f32)
        xl0 = xv0 - xh0
        xh1 = xv1.astype(jnp.bfloat16).astype(f32)
        xl1 = xv1 - xh1

        def rows(xe, xh, xl):
            c0 = (xe == 0).astype(f32)
            c1 = (xe == 1).astype(f32)
            c2 = (xe == 2).astype(f32)
            return [c0, c1, c2, xh, c0, c1, c2, xh, xl]

        feats = jnp.stack(rows(xe0, xh0, xl0) + rows(xe1, xh1, xl1),
                          axis=1)
        e3 = jax.lax.dot_general(
            feats, w18,
            dimension_numbers=(((1,), (0,)), ((), ())))

        xnc = jnp.concatenate([xnc_ref[q], xnc_ref[q + 1]], axis=1)
        h = xnc @ wco2 + ne2[0][None, :]

        a_src = h @ ws2
        a_dst = h @ wsd2 + bm2[0][None, :]
        e = e3.reshape(N * N, 2 * H)
        pre = (e @ we2).reshape(N, N, 2 * H)
        pre = pre + a_src[:, None, :] + a_dst[None, :, :]
        m = jnp.maximum(pre, 0.0)
        gm = (0.5 * jnp.tanh(e3 * 0.5) + 0.5) * m
        agg = jnp.sum(gm, axis=0)

        h_new = jnp.maximum(h + agg @ wn2 + bn2[0][None, :], 0.0)

        t_src = h_new @ wcs2
        t_dst = h_new @ wcd2
        e_new = jnp.maximum(e3 + m, 0.0)
        y = (e_new.reshape(N * N, 2 * H) @ wce2).reshape(N, N, 4)
        y = y + t_src[:, None, :] + t_dst[None, :, :]
        out_ref[q] = y[:, :, 0:2].reshape(N, 2 * N)
        out_ref[q + 1] = y[:, :, 2:4].reshape(N, 2 * N)


@jax.jit
def kernel(x_edges, x_edges_values, x_nodes, x_nodes_coord, edge_embed,
           node_embed, W_dist, W_coord, W_msg, b_msg, W_node, b_node,
           W_cls, b_cls, logit_noedge):
    B, N = x_nodes.shape
    H = node_embed.shape[1]
    C = W_cls.shape[1]
    P = B // 4

    full = lambda shape: pl.BlockSpec(shape, lambda p: (0,) * len(shape))
    out = pl.pallas_call(
        _gcn_fused_kernel,
        grid=(P,),
        in_specs=[
            pl.BlockSpec((4, N, N), lambda p: (p, 0, 0)),
            pl.BlockSpec((4, N, N), lambda p: (p, 0, 0)),
            pl.BlockSpec((4, N, 2), lambda p: (p, 0, 0)),
            full((3, H)),
            full((1, H)),
            full((1, H)),
            full((2, H)),
            full((3 * H, H)),
            full((1, H)),
            full((H, H)),
            full((1, H)),
            full((3 * H, C)),
            full((1, C)),
        ],
        out_specs=pl.BlockSpec((4, N, N * C), lambda p: (p, 0, 0)),
        out_shape=jax.ShapeDtypeStruct((B, N, N * C), jnp.float32),
    )(x_edges, x_edges_values, x_nodes_coord, edge_embed, node_embed,
      W_dist, W_coord, W_msg, b_msg.reshape(1, H), W_node,
      b_node.reshape(1, H), W_cls, b_cls.reshape(1, C))
    return out.reshape(B, N, N, C)


# speedup vs baseline: 1.0023x; 1.0023x over previous
import jax
import jax.numpy as jnp
from jax.experimental import pallas as pl
from jax.experimental.pallas import tpu as pltpu


def _bdiag(w):
    z = jnp.zeros_like(w)
    return jnp.concatenate([jnp.concatenate([w, z], axis=1),
                            jnp.concatenate([z, w], axis=1)], axis=0)


def _tile2(v):
    return jnp.concatenate([v, v], axis=1)


def _gcn_fused_kernel(xe_ref, xev_ref, xnc_ref, ee_ref, ne_ref, wd_ref,
                      wco_ref, wm_ref, bm_ref, wn_ref, bn_ref, wc_ref,
                      bc_ref, out_ref):
    N = xe_ref.shape[1]
    H = ne_ref.shape[1]
    f32 = jnp.float32

    tab = jnp.concatenate([ee_ref[...], wd_ref[...]], axis=0)
    tab_hi = tab.astype(jnp.bfloat16).astype(jnp.float32)
    tab_lo = tab - tab_hi
    wd_hi = tab_hi[3:4]
    w9 = jnp.concatenate([tab_hi, tab_lo, wd_hi], axis=0)
    w18 = _bdiag(w9)
    wco2 = _bdiag(wco_ref[...])
    wm = wm_ref[...]
    we2 = _bdiag(wm[:H])
    ws2 = _bdiag(wm[H:2 * H])
    wsd2 = _bdiag(wm[2 * H:])
    bm2 = _tile2(bm_ref[...])
    wn2 = _bdiag(wn_ref[...])
    bn2 = _tile2(bn_ref[...])
    wc = wc_ref[...]
    wce2 = _bdiag(wc[:H])
    wcs2 = _bdiag(wc[H:2 * H])
    wcd2 = _bdiag(wc[2 * H:]) + _tile2(bc_ref[...])
    ne2 = _tile2(ne_ref[...])

    xe0 = xe_ref[0]
    xe1 = xe_ref[1]
    xv0 = xev_ref[0]
    xv1 = xev_ref[1]
    xh0 = xv0.astype(jnp.bfloat16).astype(f32)
    xl0 = xv0 - xh0
    xh1 = xv1.astype(jnp.bfloat16).astype(f32)
    xl1 = xv1 - xh1

    def rows(xe, xh, xl):
        c0 = (xe == 0).astype(f32)
        c1 = (xe == 1).astype(f32)
        c2 = (xe == 2).astype(f32)
        return [c0, c1, c2, xh, c0, c1, c2, xh, xl]

    feats = jnp.stack(rows(xe0, xh0, xl0) + rows(xe1, xh1, xl1),
                      axis=1)
    e3 = jax.lax.dot_general(
        feats, w18,
        dimension_numbers=(((1,), (0,)), ((), ())))

    xnc = jnp.concatenate([xnc_ref[0], xnc_ref[1]], axis=1)
    h = xnc @ wco2 + ne2[0][None, :]

    a_src = h @ ws2
    a_dst = h @ wsd2 + bm2[0][None, :]
    e = e3.reshape(N * N, 2 * H)
    pre = (e @ we2).reshape(N, N, 2 * H)
    pre = pre + a_src[:, None, :] + a_dst[None, :, :]
    m = jnp.maximum(pre, 0.0)
    gm = (0.5 * jnp.tanh(e3 * 0.5) + 0.5) * m
    agg = jnp.sum(gm, axis=0)

    h_new = jnp.maximum(h + agg @ wn2 + bn2[0][None, :], 0.0)

    t_src = h_new @ wcs2
    t_dst = h_new @ wcd2
    e_new = jnp.maximum(e3 + m, 0.0)
    y = (e_new.reshape(N * N, 2 * H) @ wce2).reshape(N, N, 4)
    y = y + t_src[:, None, :] + t_dst[None, :, :]
    out_ref[0] = y[:, :, 0:2].reshape(N, 2 * N)
    out_ref[1] = y[:, :, 2:4].reshape(N, 2 * N)


@jax.jit
def kernel(x_edges, x_edges_values, x_nodes, x_nodes_coord, edge_embed,
           node_embed, W_dist, W_coord, W_msg, b_msg, W_node, b_node,
           W_cls, b_cls, logit_noedge):
    B, N = x_nodes.shape
    H = node_embed.shape[1]
    C = W_cls.shape[1]
    P = B // 2

    full = lambda shape: pl.BlockSpec(shape, lambda p: (0,) * len(shape))
    out = pl.pallas_call(
        _gcn_fused_kernel,
        grid=(P,),
        in_specs=[
            pl.BlockSpec((2, N, N), lambda p: (p, 0, 0)),
            pl.BlockSpec((2, N, N), lambda p: (p, 0, 0)),
            pl.BlockSpec((2, N, 2), lambda p: (p, 0, 0)),
            full((3, H)),
            full((1, H)),
            full((1, H)),
            full((2, H)),
            full((3 * H, H)),
            full((1, H)),
            full((H, H)),
            full((1, H)),
            full((3 * H, C)),
            full((1, C)),
        ],
        out_specs=pl.BlockSpec((2, N, N * C), lambda p: (p, 0, 0)),
        out_shape=jax.ShapeDtypeStruct((B, N, N * C), jnp.float32),
        compiler_params=pltpu.CompilerParams(
            dimension_semantics=("parallel",)),
    )(x_edges, x_edges_values, x_nodes_coord, edge_embed, node_embed,
      W_dist, W_coord, W_msg, b_msg.reshape(1, H), W_node,
      b_node.reshape(1, H), W_cls, b_cls.reshape(1, C))
    return out.reshape(B, N, N, C)


# speedup vs baseline: 1.0954x; 1.0929x over previous
import jax
import jax.numpy as jnp
from jax.experimental import pallas as pl
from jax.experimental.pallas import tpu as pltpu


def _bdiag(w):
    z = jnp.zeros_like(w)
    return jnp.concatenate([jnp.concatenate([w, z], axis=1),
                            jnp.concatenate([z, w], axis=1)], axis=0)


def _tile2(v):
    return jnp.concatenate([v, v], axis=1)


def _gcn_fused_kernel(xe_ref, xev_ref, xnc_ref, ee_ref, ne_ref, wd_ref,
                      wco_ref, wm_ref, bm_ref, wn_ref, bn_ref, wc_ref,
                      bc_ref, out_ref):
    N = xe_ref.shape[1]
    H = ne_ref.shape[1]
    f32 = jnp.float32

    tab = jnp.concatenate([ee_ref[...], wd_ref[...]], axis=0)
    tab_hi = tab.astype(jnp.bfloat16).astype(jnp.float32)
    tab_lo = tab - tab_hi
    wd_hi = tab_hi[3:4]
    w9 = jnp.concatenate([tab_hi, tab_lo, wd_hi], axis=0)
    w18 = _bdiag(w9)
    wco2 = _bdiag(wco_ref[...])
    wm = wm_ref[...]
    we2 = _bdiag(wm[:H])
    ws2 = _bdiag(wm[H:2 * H])
    wsd2 = _bdiag(wm[2 * H:])
    bm2 = _tile2(bm_ref[...])
    wn2 = _bdiag(wn_ref[...])
    bn2 = _tile2(bn_ref[...])
    wc = wc_ref[...]
    wce2 = _bdiag(wc[:H])
    wcs2 = _bdiag(wc[H:2 * H])
    wcd2 = _bdiag(wc[2 * H:]) + _tile2(bc_ref[...])
    ne2 = _tile2(ne_ref[...])

    xe0 = xe_ref[0]
    xe1 = xe_ref[1]
    xv0 = xev_ref[0]
    xv1 = xev_ref[1]
    xh0 = xv0.astype(jnp.bfloat16).astype(f32)
    xl0 = xv0 - xh0
    xh1 = xv1.astype(jnp.bfloat16).astype(f32)
    xl1 = xv1 - xh1

    def rows(xe, xh, xl):
        c0 = (xe == 0).astype(f32)
        c1 = (xe == 1).astype(f32)
        c2 = (xe == 2).astype(f32)
        return [c0, c1, c2, xh, c0, c1, c2, xh, xl]

    feats = jnp.stack(rows(xe0, xh0, xl0) + rows(xe1, xh1, xl1),
                      axis=1)
    e3 = jax.lax.dot_general(
        feats, w18,
        dimension_numbers=(((1,), (0,)), ((), ())))

    xnc = jnp.concatenate([xnc_ref[0], xnc_ref[1]], axis=1)
    h = xnc @ wco2 + ne2[0][None, :]

    a_src = h @ ws2
    a_dst = h @ wsd2 + bm2[0][None, :]
    e = e3.reshape(N * N, 2 * H)
    pre = (e @ we2).reshape(N, N, 2 * H)
    pre = pre + a_src[:, None, :] + a_dst[None, :, :]
    m = jnp.maximum(pre, 0.0)
    gm = (0.5 * jnp.tanh(e3 * 0.5) + 0.5) * m
    agg = jnp.sum(gm, axis=0)

    h_new = jnp.maximum(h + agg @ wn2 + bn2[0][None, :], 0.0)

    t_src = h_new @ wcs2
    t_dst = h_new @ wcd2
    e_new = jnp.maximum(e3 + m, 0.0)
    y = (e_new.reshape(N * N, 2 * H) @ wce2).reshape(N, N, 4)
    y = y + t_src[:, None, :] + t_dst[None, :, :]
    i_l = jax.lax.broadcasted_iota(jnp.int32, (N, 2 * N), 1)
    i_j = jax.lax.broadcasted_iota(jnp.int32, (N, 2 * N), 0)
    perm = (i_l == 2 * i_j).astype(f32)
    dims = (((1,), (0,)), ((), ()))
    yh = y.astype(jnp.bfloat16).astype(f32)
    yl = y - yh
    z = (jax.lax.dot_general(yh, perm, dimension_numbers=dims)
         + jax.lax.dot_general(yl, perm, dimension_numbers=dims))
    out_ref[0] = z[:, 0, :] + jnp.roll(z[:, 1, :], 1, axis=1)
    out_ref[1] = z[:, 2, :] + jnp.roll(z[:, 3, :], 1, axis=1)


@jax.jit
def kernel(x_edges, x_edges_values, x_nodes, x_nodes_coord, edge_embed,
           node_embed, W_dist, W_coord, W_msg, b_msg, W_node, b_node,
           W_cls, b_cls, logit_noedge):
    B, N = x_nodes.shape
    H = node_embed.shape[1]
    C = W_cls.shape[1]
    P = B // 2

    full = lambda shape: pl.BlockSpec(shape, lambda p: (0,) * len(shape))
    out = pl.pallas_call(
        _gcn_fused_kernel,
        grid=(P,),
        in_specs=[
            pl.BlockSpec((2, N, N), lambda p: (p, 0, 0)),
            pl.BlockSpec((2, N, N), lambda p: (p, 0, 0)),
            pl.BlockSpec((2, N, 2), lambda p: (p, 0, 0)),
            full((3, H)),
            full((1, H)),
            full((1, H)),
            full((2, H)),
            full((3 * H, H)),
            full((1, H)),
            full((H, H)),
            full((1, H)),
            full((3 * H, C)),
            full((1, C)),
        ],
        out_specs=pl.BlockSpec((2, N, N * C), lambda p: (p, 0, 0)),
        out_shape=jax.ShapeDtypeStruct((B, N, N * C), jnp.float32),
        compiler_params=pltpu.CompilerParams(
            dimension_semantics=("parallel",)),
    )(x_edges, x_edges_values, x_nodes_coord, edge_embed, node_embed,
      W_dist, W_coord, W_msg, b_msg.reshape(1, H), W_node,
      b_node.reshape(1, H), W_cls, b_cls.reshape(1, C))
    return out.reshape(B, N, N, C)
